# Initial kernel scaffold; baseline (speedup 1.0000x reference)
import jax
import jax.numpy as jnp
from jax.experimental import pallas as pl


def kernel(x, W_wave, b_wave, W_kernel, b_kernel, W_out):
    raise NotImplementedError("write your pallas kernel here")


# speedup vs baseline: 1.8324x; 1.8324x over previous
import functools

import jax
import jax.numpy as jnp
from jax import lax
from jax.experimental import pallas as pl
from jax.experimental.pallas import tpu as pltpu
from jax.experimental.pallas import tpu_sc as plsc

L = 2048
C = 4096
H = 32
K = 64
S = 33
SP = 40
NCH = 4
CHC = C // NCH
HPC = H // NCH
D = C // H
LBLK = 256
MAX_FREQ = 16.0
MIN_FREQ = 1.0
MAX_RECEPTIVE = 16.0 * MAX_FREQ

NW = 32
TPW = L // NW


def _silu(v):
    return v * jax.nn.sigmoid(v)


def _wave_body(x_ref, ww_ref, bw_ref, rel_ref, idx_ref):
    i = pl.program_id(0)
    wv = jnp.dot(x_ref[...], ww_ref[...], preferred_element_type=jnp.float32)
    wave = _silu(wv + bw_ref[...])
    freq = jax.nn.sigmoid(wave[:, :H]) * (MAX_FREQ - MIN_FREQ) + MIN_FREQ
    phase = jnp.tanh(wave[:, H:]) * MAX_FREQ
    freq_avg = jnp.mean(freq, axis=1, keepdims=True)
    phase_avg = jnp.mean(phase, axis=1, keepdims=True)
    offs = lax.broadcasted_iota(jnp.int32, (LBLK, SP), 1).astype(
        jnp.float32) - 16.0
    rel = offs * freq_avg + phase_avg
    rel_ref[...] = rel
    center = (i * LBLK).astype(jnp.float32) + lax.broadcasted_iota(
        jnp.int32, (LBLK, SP), 0).astype(jnp.float32)
    spos = center + rel
    sidx = jnp.clip(spos.astype(jnp.int32), 0, L - 1)
    i4 = sidx * NCH
    for c in range(NCH):
        idx_ref[c] = i4 + c


def _stage_a(x2, w_wave, b_wave2):
    return pl.pallas_call(
        _wave_body,
        grid=(L // LBLK,),
        in_specs=[
            pl.BlockSpec((LBLK, C), lambda i: (i, 0)),
            pl.BlockSpec((C, 2 * H), lambda i: (0, 0)),
            pl.BlockSpec((1, 2 * H), lambda i: (0, 0)),
        ],
        out_specs=[
            pl.BlockSpec((LBLK, SP), lambda i: (i, 0)),
            pl.BlockSpec((NCH, LBLK, SP), lambda i: (0, i, 0)),
        ],
        out_shape=[
            jax.ShapeDtypeStruct((L, SP), jnp.float32),
            jax.ShapeDtypeStruct((NCH, L, SP), jnp.int32),
        ],
    )(x2, w_wave, b_wave2)


LBLKB = 128


def _weights_body(x_ref, wk_ref, bk_ref, rel_ref, w_ref):
    i = pl.program_id(1)
    km = _silu(jnp.dot(x_ref[...], wk_ref[...],
                       preferred_element_type=jnp.float32) + bk_ref[...])
    km3 = km.reshape(LBLKB, HPC, K)
    rel = rel_ref[...]
    center = (i * LBLKB).astype(jnp.float32) + lax.broadcasted_iota(
        jnp.int32, (LBLKB, SP), 0).astype(jnp.float32)
    spos = center + rel
    validf = ((spos >= 0.0) & (spos < float(L))).astype(jnp.float32)
    npos = jnp.clip((rel + MAX_RECEPTIVE) / (2.0 * MAX_RECEPTIVE), 0.0, 1.0)
    idxf = npos * (K - 1)
    ifl = jnp.clip(idxf.astype(jnp.int32), 0, K - 2)
    wce = idxf - ifl.astype(jnp.float32)
    wfl = 1.0 - wce
    iota_k = lax.broadcasted_iota(jnp.int32, (LBLKB, K), 1)
    pieces = []
    den = jnp.zeros((LBLKB, HPC), jnp.float32)
    for s in range(S):
        m_f = (iota_k == ifl[:, s:s + 1]).astype(jnp.float32)
        m_c = (iota_k == (ifl[:, s:s + 1] + 1)).astype(jnp.float32)
        wsel = wfl[:, s:s + 1] * m_f + wce[:, s:s + 1] * m_c
        ks = jnp.sum(km3 * wsel[:, None, :], axis=2)
        ks = ks * validf[:, s:s + 1]
        den = den + ks
        pieces.append(ks)
    den = den + 1e-8
    out = jnp.concatenate(
        [p / den for p in pieces]
        + [jnp.zeros((LBLKB, (SP - S) * HPC), jnp.float32)], axis=1)
    w_ref[0] = out


def _stage_b(x2, w_kernel, b_kernel2, rel):
    return pl.pallas_call(
        _weights_body,
        grid=(NCH, L // LBLKB),
        in_specs=[
            pl.BlockSpec((LBLKB, C), lambda j, i: (i, 0)),
            pl.BlockSpec((C, H * K // NCH), lambda j, i: (0, j)),
            pl.BlockSpec((1, H * K // NCH), lambda j, i: (0, j)),
            pl.BlockSpec((LBLKB, SP), lambda j, i: (i, 0)),
        ],
        out_specs=pl.BlockSpec((1, LBLKB, SP * HPC), lambda j, i: (j, i, 0)),
        out_shape=jax.ShapeDtypeStruct((NCH, L, SP * HPC), jnp.float32),
    )(x2, w_kernel, b_kernel2, rel)


def _sc_gather(xr, idx4, warr):
    mesh = plsc.VectorSubcoreMesh(core_axis_name="c", subcore_axis_name="s")

    @functools.partial(
        pl.kernel,
        mesh=mesh,
        out_type=jax.ShapeDtypeStruct((L, C), jnp.float32),
        scratch_types=[
            pltpu.VMEM((SP,), jnp.int32),
            pltpu.VMEM((SP, CHC), jnp.float32),
            pltpu.VMEM((SP * HPC,), jnp.float32),
            pltpu.VMEM((CHC,), jnp.float32),
            pltpu.SemaphoreType.DMA,
        ],
        compiler_params=pltpu.CompilerParams(needs_layout_passes=False),
    )
    def body(xr_hbm, idx_hbm, w_hbm, out_hbm, idx_v, rows_v, w_v, acc_v, sem):
        wid = lax.axis_index("s") * 2 + lax.axis_index("c")

        def token_body(t, carry):
            l = wid * TPW + t
            for c in range(NCH):
                pltpu.sync_copy(idx_hbm.at[c, l], idx_v)
                gather = pltpu.async_copy(xr_hbm.at[idx_v], rows_v, sem)
                pltpu.sync_copy(w_hbm.at[c, l], w_v)
                gather.wait()
                for h in range(HPC):
                    def s_body(s, accs):
                        wspl = plsc.load_gather(
                            w_v, [jnp.full((16,), s * HPC + h, jnp.int32)])
                        return tuple(
                            accs[j] + wspl * rows_v[s, pl.ds(h * D + j * 16, 16)]
                            for j in range(8))
                    accs = lax.fori_loop(
                        0, SP, s_body,
                        tuple(jnp.zeros((16,), jnp.float32) for _ in range(8)))
                    for j in range(8):
                        acc_v[pl.ds(h * D + j * 16, 16)] = accs[j]
                pltpu.sync_copy(acc_v, out_hbm.at[l, pl.ds(c * CHC, CHC)])
            return carry

        lax.fori_loop(0, TPW, token_body, 0)

    return body(xr, idx4, warr)


def _out_body(g_ref, wo_ref, o_ref):
    acc = jnp.dot(g_ref[...], wo_ref[...], preferred_element_type=jnp.float32)
    o_ref[...] = _silu(acc)


def _stage_d(og, w_out):
    nj = 8
    return pl.pallas_call(
        _out_body,
        grid=(nj, L // LBLK),
        in_specs=[
            pl.BlockSpec((LBLK, C), lambda j, i: (i, 0)),
            pl.BlockSpec((C, C // nj), lambda j, i: (0, j)),
        ],
        out_specs=pl.BlockSpec((LBLK, C // nj), lambda j, i: (i, j)),
        out_shape=jax.ShapeDtypeStruct((L, C), jnp.float32),
    )(og, w_out)


def kernel(x, W_wave, b_wave, W_kernel, b_kernel, W_out):
    x2 = x[0]
    xr = x2.reshape(L * NCH, CHC)
    rel, idx4 = _stage_a(x2, W_wave, b_wave.reshape(1, -1))
    warr = _stage_b(x2, W_kernel, b_kernel.reshape(1, -1), rel)
    og = _sc_gather(xr, idx4, warr)
    y = _stage_d(og, W_out)
    return y[None]


# speedup vs baseline: 2.4380x; 1.3305x over previous
import functools

import jax
import jax.numpy as jnp
from jax import lax
from jax.experimental import pallas as pl
from jax.experimental.pallas import tpu as pltpu
from jax.experimental.pallas import tpu_sc as plsc

L = 2048
C = 4096
H = 32
K = 64
S = 33
SP = 40
NCH = 4
CHC = C // NCH
HPC = H // NCH
D = C // H
LBLK = 256
MAX_FREQ = 16.0
MIN_FREQ = 1.0
MAX_RECEPTIVE = 16.0 * MAX_FREQ

NW = 32
TPW = L // NW


def _silu(v):
    return v * jax.nn.sigmoid(v)


def _wave_body(x_ref, ww_ref, bw_ref, rel_ref, idx_ref):
    i = pl.program_id(0)
    wv = jnp.dot(x_ref[...], ww_ref[...], preferred_element_type=jnp.float32)
    wave = _silu(wv + bw_ref[...])
    freq = jax.nn.sigmoid(wave[:, :H]) * (MAX_FREQ - MIN_FREQ) + MIN_FREQ
    phase = jnp.tanh(wave[:, H:]) * MAX_FREQ
    freq_avg = jnp.mean(freq, axis=1, keepdims=True)
    phase_avg = jnp.mean(phase, axis=1, keepdims=True)
    offs = lax.broadcasted_iota(jnp.int32, (LBLK, SP), 1).astype(
        jnp.float32) - 16.0
    rel = offs * freq_avg + phase_avg
    rel_ref[...] = rel
    center = (i * LBLK).astype(jnp.float32) + lax.broadcasted_iota(
        jnp.int32, (LBLK, SP), 0).astype(jnp.float32)
    spos = center + rel
    sidx = jnp.clip(spos.astype(jnp.int32), 0, L - 1)
    i4 = sidx * NCH
    for c in range(NCH):
        idx_ref[:, c, :] = i4 + c


def _stage_a(x2, w_wave, b_wave2):
    return pl.pallas_call(
        _wave_body,
        grid=(L // LBLK,),
        in_specs=[
            pl.BlockSpec((LBLK, C), lambda i: (i, 0)),
            pl.BlockSpec((C, 2 * H), lambda i: (0, 0)),
            pl.BlockSpec((1, 2 * H), lambda i: (0, 0)),
        ],
        out_specs=[
            pl.BlockSpec((LBLK, SP), lambda i: (i, 0)),
            pl.BlockSpec((LBLK, NCH, SP), lambda i: (i, 0, 0)),
        ],
        out_shape=[
            jax.ShapeDtypeStruct((L, SP), jnp.float32),
            jax.ShapeDtypeStruct((L, NCH, SP), jnp.int32),
        ],
    )(x2, w_wave, b_wave2)


LBLKB = 128


def _weights_body(x_ref, wk_ref, bk_ref, rel_ref, w_ref):
    i = pl.program_id(1)
    km = _silu(jnp.dot(x_ref[...], wk_ref[...],
                       preferred_element_type=jnp.float32) + bk_ref[...])
    km3 = km.reshape(LBLKB, HPC, K)
    rel = rel_ref[...]
    center = (i * LBLKB).astype(jnp.float32) + lax.broadcasted_iota(
        jnp.int32, (LBLKB, SP), 0).astype(jnp.float32)
    spos = center + rel
    validf = ((spos >= 0.0) & (spos < float(L))).astype(jnp.float32)
    npos = jnp.clip((rel + MAX_RECEPTIVE) / (2.0 * MAX_RECEPTIVE), 0.0, 1.0)
    idxf = npos * (K - 1)
    ifl = jnp.clip(idxf.astype(jnp.int32), 0, K - 2)
    wce = idxf - ifl.astype(jnp.float32)
    wfl = 1.0 - wce
    iota_k = lax.broadcasted_iota(jnp.int32, (LBLKB, K), 1)
    pieces = []
    den = jnp.zeros((LBLKB, HPC), jnp.float32)
    for s in range(S):
        m_f = (iota_k == ifl[:, s:s + 1]).astype(jnp.float32)
        m_c = (iota_k == (ifl[:, s:s + 1] + 1)).astype(jnp.float32)
        wsel = wfl[:, s:s + 1] * m_f + wce[:, s:s + 1] * m_c
        ks = jnp.sum(km3 * wsel[:, None, :], axis=2)
        ks = ks * validf[:, s:s + 1]
        den = den + ks
        pieces.append(ks)
    den = den + 1e-8
    out = jnp.concatenate(
        [p / den for p in pieces]
        + [jnp.zeros((LBLKB, (SP - S) * HPC), jnp.float32)], axis=1)
    w_ref[:, 0, 0, :] = out


def _stage_b(x2, w_kernel, b_kernel2, rel):
    return pl.pallas_call(
        _weights_body,
        grid=(NCH, L // LBLKB),
        in_specs=[
            pl.BlockSpec((LBLKB, C), lambda j, i: (i, 0)),
            pl.BlockSpec((C, H * K // NCH), lambda j, i: (0, j)),
            pl.BlockSpec((1, H * K // NCH), lambda j, i: (0, j)),
            pl.BlockSpec((LBLKB, SP), lambda j, i: (i, 0)),
        ],
        out_specs=pl.BlockSpec((LBLKB, 1, 1, SP * HPC),
                               lambda j, i: (i, j, 0, 0)),
        out_shape=jax.ShapeDtypeStruct((L, NCH, 1, SP * HPC), jnp.float32),
    )(x2, w_kernel, b_kernel2, rel)


NTASK = TPW * NCH


def _sc_compute(rows_v, w_v, acc_v):
    for h in range(HPC):
        def s_body(s, accs):
            wspl = plsc.load_gather(
                w_v, [jnp.full((16,), s * HPC + h, jnp.int32)])
            return tuple(
                accs[j] + wspl * rows_v[s, pl.ds(h * D + j * 16, 16)]
                for j in range(8))
        accs = lax.fori_loop(
            0, SP, s_body,
            tuple(jnp.zeros((16,), jnp.float32) for _ in range(8)))
        for j in range(8):
            acc_v[pl.ds(h * D + j * 16, 16)] = accs[j]


def _sc_gather(xr, idx4, warr):
    mesh = plsc.VectorSubcoreMesh(core_axis_name="c", subcore_axis_name="s")

    @functools.partial(
        pl.kernel,
        mesh=mesh,
        out_type=jax.ShapeDtypeStruct((L, NCH, CHC), jnp.float32),
        scratch_types=[
            pltpu.VMEM((SP,), jnp.int32),
            pltpu.VMEM((SP,), jnp.int32),
            pltpu.VMEM((SP, CHC), jnp.float32),
            pltpu.VMEM((SP, CHC), jnp.float32),
            pltpu.VMEM((SP * HPC,), jnp.float32),
            pltpu.VMEM((SP * HPC,), jnp.float32),
            pltpu.VMEM((CHC,), jnp.float32),
            pltpu.VMEM((CHC,), jnp.float32),
            pltpu.SemaphoreType.DMA,
            pltpu.SemaphoreType.DMA,
            pltpu.SemaphoreType.DMA,
            pltpu.SemaphoreType.DMA,
            pltpu.SemaphoreType.DMA,
            pltpu.SemaphoreType.DMA,
            pltpu.SemaphoreType.DMA,
            pltpu.SemaphoreType.DMA,
        ],
        compiler_params=pltpu.CompilerParams(needs_layout_passes=False),
    )
    def body(xr_hbm, idx_hbm, w_hbm, out_hbm,
             idx_a, idx_b, rows0, rows1, w0, w1, acc0, acc1,
             s_ia, s_ib, s_r0, s_r1, s_w0, s_w1, s_s0, s_s1):
        wid = lax.axis_index("s") * 2 + lax.axis_index("c")
        base = wid * TPW

        def lc(task):
            return base + (task >> 2), task & 3

        l0, c0 = lc(0)
        pltpu.sync_copy(idx_hbm.at[l0, c0], idx_a)
        pltpu.async_copy(xr_hbm.at[idx_a], rows0, s_r0)
        l1, c1 = lc(1)
        pltpu.async_copy(idx_hbm.at[l1, c1], idx_b, s_ib)
        pltpu.async_copy(w_hbm.at[l0, c0, 0], w0, s_w0)

        def pair_body(p, carry):
            t_a = 2 * p
            t_b = t_a + 1
            t_c = t_a + 2
            la, ca = lc(t_a)
            lb, cb = lc(t_b)
            lcl, ccl = lc(jnp.minimum(t_c, NTASK - 1))
            pltpu.make_async_copy(idx_hbm.at[lb, cb], idx_b, s_ib).wait()
            pltpu.async_copy(xr_hbm.at[idx_b], rows1, s_r1)
            pltpu.async_copy(w_hbm.at[lb, cb, 0], w1, s_w1)
            pltpu.make_async_copy(w_hbm.at[la, ca, 0], w0, s_w0).wait()
            pltpu.make_async_copy(xr_hbm.at[idx_b], rows0, s_r0).wait()
            pltpu.async_copy(idx_hbm.at[lcl, ccl], idx_a, s_ia)

            @pl.when(p > 0)
            def _():
                pltpu.make_async_copy(acc0, out_hbm.at[la, ca], s_s0).wait()
            _sc_compute(rows0, w0, acc0)
            pltpu.async_copy(acc0, out_hbm.at[la, ca], s_s0)
            pltpu.make_async_copy(idx_hbm.at[lcl, ccl], idx_a, s_ia).wait()

            @pl.when(t_c < NTASK)
            def _():
                pltpu.async_copy(xr_hbm.at[idx_a], rows0, s_r0)
                pltpu.async_copy(w_hbm.at[lcl, ccl, 0], w0, s_w0)
            pltpu.make_async_copy(w_hbm.at[lb, cb, 0], w1, s_w1).wait()
            pltpu.make_async_copy(xr_hbm.at[idx_b], rows1, s_r1).wait()

            @pl.when(t_c < NTASK)
            def _():
                ld, cd = lc(jnp.minimum(t_c + 1, NTASK - 1))
                pltpu.async_copy(idx_hbm.at[ld, cd], idx_b, s_ib)

            @pl.when(p > 0)
            def _():
                pltpu.make_async_copy(acc1, out_hbm.at[lb, cb], s_s1).wait()
            _sc_compute(rows1, w1, acc1)
            pltpu.async_copy(acc1, out_hbm.at[lb, cb], s_s1)
            return carry

        lax.fori_loop(0, NTASK // 2, pair_body, 0)
        lf, cf = lc(NTASK - 2)
        pltpu.make_async_copy(acc0, out_hbm.at[lf, cf], s_s0).wait()
        lg, cg = lc(NTASK - 1)
        pltpu.make_async_copy(acc1, out_hbm.at[lg, cg], s_s1).wait()

    return body(xr, idx4, warr)


def _out_body(g_ref, wo_ref, o_ref):
    acc = jnp.dot(g_ref[...], wo_ref[...], preferred_element_type=jnp.float32)
    o_ref[...] = _silu(acc)


def _stage_d(og, w_out):
    nj = 8
    return pl.pallas_call(
        _out_body,
        grid=(nj, L // LBLK),
        in_specs=[
            pl.BlockSpec((LBLK, C), lambda j, i: (i, 0)),
            pl.BlockSpec((C, C // nj), lambda j, i: (0, j)),
        ],
        out_specs=pl.BlockSpec((LBLK, C // nj), lambda j, i: (i, j)),
        out_shape=jax.ShapeDtypeStruct((L, C), jnp.float32),
    )(og, w_out)


def kernel(x, W_wave, b_wave, W_kernel, b_kernel, W_out):
    x2 = x[0]
    xr = x2.reshape(L * NCH, CHC)
    rel, idx4 = _stage_a(x2, W_wave, b_wave.reshape(1, -1))
    warr = _stage_b(x2, W_kernel, b_kernel.reshape(1, -1), rel)
    og = _sc_gather(xr, idx4, warr).reshape(L, C)
    y = _stage_d(og, W_out)
    return y[None]


# speedup vs baseline: 4.1711x; 1.7109x over previous
import functools

import jax
import jax.numpy as jnp
from jax import lax
from jax.experimental import pallas as pl
from jax.experimental.pallas import tpu as pltpu
from jax.experimental.pallas import tpu_sc as plsc

L = 2048
C = 4096
H = 32
K = 64
S = 33
SP = 40
NCH = 4
CHC = C // NCH
HPC = H // NCH
D = C // H
LBLK = 256
MAX_FREQ = 16.0
MIN_FREQ = 1.0
MAX_RECEPTIVE = 16.0 * MAX_FREQ

NW = 32
TPW = L // NW


def _silu(v):
    return v * jax.nn.sigmoid(v)


def _wave_body(x_ref, ww_ref, bw_ref, rel_ref, idx_ref):
    i = pl.program_id(0)
    wv = jnp.dot(x_ref[...], ww_ref[...], preferred_element_type=jnp.float32)
    wave = _silu(wv + bw_ref[...])
    freq = jax.nn.sigmoid(wave[:, :H]) * (MAX_FREQ - MIN_FREQ) + MIN_FREQ
    phase = jnp.tanh(wave[:, H:]) * MAX_FREQ
    freq_avg = jnp.mean(freq, axis=1, keepdims=True)
    phase_avg = jnp.mean(phase, axis=1, keepdims=True)
    offs = lax.broadcasted_iota(jnp.int32, (LBLK, SP), 1).astype(
        jnp.float32) - 16.0
    rel = offs * freq_avg + phase_avg
    rel_ref[...] = rel
    center = (i * LBLK).astype(jnp.float32) + lax.broadcasted_iota(
        jnp.int32, (LBLK, SP), 0).astype(jnp.float32)
    spos = center + rel
    sidx = jnp.clip(spos.astype(jnp.int32), 0, L - 1)
    i4 = sidx * NCH
    for c in range(NCH):
        idx_ref[:, c, :] = i4 + c


def _stage_a(x2, w_wave, b_wave2):
    return pl.pallas_call(
        _wave_body,
        grid=(L // LBLK,),
        in_specs=[
            pl.BlockSpec((LBLK, C), lambda i: (i, 0)),
            pl.BlockSpec((C, 2 * H), lambda i: (0, 0)),
            pl.BlockSpec((1, 2 * H), lambda i: (0, 0)),
        ],
        out_specs=[
            pl.BlockSpec((LBLK, SP), lambda i: (i, 0)),
            pl.BlockSpec((LBLK, NCH, SP), lambda i: (i, 0, 0)),
        ],
        out_shape=[
            jax.ShapeDtypeStruct((L, SP), jnp.float32),
            jax.ShapeDtypeStruct((L, NCH, SP), jnp.int32),
        ],
    )(x2, w_wave, b_wave2)


LBLKB = 128


def _weights_body(x_ref, wk_ref, bk_ref, rel_ref, w_ref):
    i = pl.program_id(1)
    km = _silu(jnp.dot(x_ref[...], wk_ref[...],
                       preferred_element_type=jnp.float32) + bk_ref[...])
    km3 = km.reshape(LBLKB, HPC, K)
    rel = rel_ref[...]
    center = (i * LBLKB).astype(jnp.float32) + lax.broadcasted_iota(
        jnp.int32, (LBLKB, SP), 0).astype(jnp.float32)
    spos = center + rel
    validf = ((spos >= 0.0) & (spos < float(L))).astype(jnp.float32)
    npos = jnp.clip((rel + MAX_RECEPTIVE) / (2.0 * MAX_RECEPTIVE), 0.0, 1.0)
    idxf = npos * (K - 1)
    ifl = jnp.clip(idxf.astype(jnp.int32), 0, K - 2)
    wce = idxf - ifl.astype(jnp.float32)
    wfl = 1.0 - wce
    pad_mask = (lax.broadcasted_iota(jnp.int32, (LBLKB, SP), 1)
                < S).astype(jnp.float32)
    validf = validf * pad_mask
    ifl3 = jnp.broadcast_to(ifl[:, None, :], (LBLKB, HPC, SP))
    kf = jnp.take_along_axis(km3, ifl3, axis=2)
    kc = jnp.take_along_axis(km3, ifl3 + 1, axis=2)
    kern = (wfl[:, None, :] * kf + wce[:, None, :] * kc) * validf[:, None, :]
    den = jnp.sum(kern, axis=2, keepdims=True) + 1e-8
    w_ref[:, 0, 0, :] = (kern / den).reshape(LBLKB, HPC * SP)


def _stage_b(x2, w_kernel, b_kernel2, rel):
    return pl.pallas_call(
        _weights_body,
        grid=(NCH, L // LBLKB),
        in_specs=[
            pl.BlockSpec((LBLKB, C), lambda j, i: (i, 0)),
            pl.BlockSpec((C, H * K // NCH), lambda j, i: (0, j)),
            pl.BlockSpec((1, H * K // NCH), lambda j, i: (0, j)),
            pl.BlockSpec((LBLKB, SP), lambda j, i: (i, 0)),
        ],
        out_specs=pl.BlockSpec((LBLKB, 1, 1, SP * HPC),
                               lambda j, i: (i, j, 0, 0)),
        out_shape=jax.ShapeDtypeStruct((L, NCH, 1, SP * HPC), jnp.float32),
    )(x2, w_kernel, b_kernel2, rel)


NTASK = TPW * NCH


def _sc_compute(rows_v, w_v, acc_v):
    for h in range(HPC):
        def s_body(s, accs):
            wspl = plsc.load_gather(
                w_v, [jnp.full((16,), h * SP, jnp.int32) + s])
            return tuple(
                accs[j] + wspl * rows_v[s, pl.ds(h * D + j * 16, 16)]
                for j in range(8))
        accs = lax.fori_loop(
            0, SP, s_body,
            tuple(jnp.zeros((16,), jnp.float32) for _ in range(8)))
        for j in range(8):
            acc_v[pl.ds(h * D + j * 16, 16)] = accs[j]


def _sc_gather(xr, idx4, warr):
    mesh = plsc.VectorSubcoreMesh(core_axis_name="c", subcore_axis_name="s")

    @functools.partial(
        pl.kernel,
        mesh=mesh,
        out_type=jax.ShapeDtypeStruct((L, NCH, CHC), jnp.float32),
        scratch_types=[
            pltpu.VMEM((SP,), jnp.int32),
            pltpu.VMEM((SP,), jnp.int32),
            pltpu.VMEM((SP, CHC), jnp.float32),
            pltpu.VMEM((SP, CHC), jnp.float32),
            pltpu.VMEM((SP * HPC,), jnp.float32),
            pltpu.VMEM((SP * HPC,), jnp.float32),
            pltpu.VMEM((CHC,), jnp.float32),
            pltpu.VMEM((CHC,), jnp.float32),
            pltpu.SemaphoreType.DMA,
            pltpu.SemaphoreType.DMA,
            pltpu.SemaphoreType.DMA,
            pltpu.SemaphoreType.DMA,
            pltpu.SemaphoreType.DMA,
            pltpu.SemaphoreType.DMA,
            pltpu.SemaphoreType.DMA,
            pltpu.SemaphoreType.DMA,
        ],
        compiler_params=pltpu.CompilerParams(needs_layout_passes=False),
    )
    def body(xr_hbm, idx_hbm, w_hbm, out_hbm,
             idx_a, idx_b, rows0, rows1, w0, w1, acc0, acc1,
             s_ia, s_ib, s_r0, s_r1, s_w0, s_w1, s_s0, s_s1):
        wid = lax.axis_index("s") * 2 + lax.axis_index("c")
        base = wid * TPW

        def lc(task):
            return base + (task >> 2), task & 3

        l0, c0 = lc(0)
        pltpu.sync_copy(idx_hbm.at[l0, c0], idx_a)
        pltpu.async_copy(xr_hbm.at[idx_a], rows0, s_r0)
        l1, c1 = lc(1)
        pltpu.async_copy(idx_hbm.at[l1, c1], idx_b, s_ib)
        pltpu.async_copy(w_hbm.at[l0, c0, 0], w0, s_w0)

        def pair_body(p, carry):
            t_a = 2 * p
            t_b = t_a + 1
            t_c = t_a + 2
            la, ca = lc(t_a)
            lb, cb = lc(t_b)
            lcl, ccl = lc(jnp.minimum(t_c, NTASK - 1))
            pltpu.make_async_copy(idx_hbm.at[lb, cb], idx_b, s_ib).wait()
            pltpu.async_copy(xr_hbm.at[idx_b], rows1, s_r1)
            pltpu.async_copy(w_hbm.at[lb, cb, 0], w1, s_w1)
            pltpu.make_async_copy(w_hbm.at[la, ca, 0], w0, s_w0).wait()
            pltpu.make_async_copy(xr_hbm.at[idx_b], rows0, s_r0).wait()
            pltpu.async_copy(idx_hbm.at[lcl, ccl], idx_a, s_ia)

            @pl.when(p > 0)
            def _():
                pltpu.make_async_copy(acc0, out_hbm.at[la, ca], s_s0).wait()
            _sc_compute(rows0, w0, acc0)
            pltpu.async_copy(acc0, out_hbm.at[la, ca], s_s0)
            pltpu.make_async_copy(idx_hbm.at[lcl, ccl], idx_a, s_ia).wait()

            @pl.when(t_c < NTASK)
            def _():
                pltpu.async_copy(xr_hbm.at[idx_a], rows0, s_r0)
                pltpu.async_copy(w_hbm.at[lcl, ccl, 0], w0, s_w0)
            pltpu.make_async_copy(w_hbm.at[lb, cb, 0], w1, s_w1).wait()
            pltpu.make_async_copy(xr_hbm.at[idx_b], rows1, s_r1).wait()

            @pl.when(t_c < NTASK)
            def _():
                ld, cd = lc(jnp.minimum(t_c + 1, NTASK - 1))
                pltpu.async_copy(idx_hbm.at[ld, cd], idx_b, s_ib)

            @pl.when(p > 0)
            def _():
                pltpu.make_async_copy(acc1, out_hbm.at[lb, cb], s_s1).wait()
            _sc_compute(rows1, w1, acc1)
            pltpu.async_copy(acc1, out_hbm.at[lb, cb], s_s1)
            return carry

        lax.fori_loop(0, NTASK // 2, pair_body, 0)
        lf, cf = lc(NTASK - 2)
        pltpu.make_async_copy(acc0, out_hbm.at[lf, cf], s_s0).wait()
        lg, cg = lc(NTASK - 1)
        pltpu.make_async_copy(acc1, out_hbm.at[lg, cg], s_s1).wait()

    return body(xr, idx4, warr)


def _out_body(g_ref, wo_ref, o_ref):
    acc = jnp.dot(g_ref[...], wo_ref[...], preferred_element_type=jnp.float32)
    o_ref[...] = _silu(acc)


def _stage_d(og, w_out):
    nj = 8
    return pl.pallas_call(
        _out_body,
        grid=(nj, L // LBLK),
        in_specs=[
            pl.BlockSpec((LBLK, C), lambda j, i: (i, 0)),
            pl.BlockSpec((C, C // nj), lambda j, i: (0, j)),
        ],
        out_specs=pl.BlockSpec((LBLK, C // nj), lambda j, i: (i, j)),
        out_shape=jax.ShapeDtypeStruct((L, C), jnp.float32),
    )(og, w_out)


def kernel(x, W_wave, b_wave, W_kernel, b_kernel, W_out):
    x2 = x[0]
    xr = x2.reshape(L * NCH, CHC)
    rel, idx4 = _stage_a(x2, W_wave, b_wave.reshape(1, -1))
    warr = _stage_b(x2, W_kernel, b_kernel.reshape(1, -1), rel)
    og = _sc_gather(xr, idx4, warr).reshape(L, C)
    y = _stage_d(og, W_out)
    return y[None]


# speedup vs baseline: 4.6037x; 1.1037x over previous
import functools

import jax
import jax.numpy as jnp
from jax import lax
from jax.experimental import pallas as pl
from jax.experimental.pallas import tpu as pltpu
from jax.experimental.pallas import tpu_sc as plsc

L = 2048
C = 4096
H = 32
K = 64
S = 33
SP = 40
NCH = 4
CHC = C // NCH
HPC = H // NCH
D = C // H
LBLK = 256
MAX_FREQ = 16.0
MIN_FREQ = 1.0
MAX_RECEPTIVE = 16.0 * MAX_FREQ

NW = 32
TPW = L // NW


def _silu(v):
    return v * jax.nn.sigmoid(v)


def _wave_body(x_ref, ww_ref, bw_ref, rel_ref, idx_ref):
    i = pl.program_id(0)
    wv = jnp.dot(x_ref[...], ww_ref[...], preferred_element_type=jnp.float32)
    wave = _silu(wv + bw_ref[...])
    freq = jax.nn.sigmoid(wave[:, :H]) * (MAX_FREQ - MIN_FREQ) + MIN_FREQ
    phase = jnp.tanh(wave[:, H:]) * MAX_FREQ
    freq_avg = jnp.mean(freq, axis=1, keepdims=True)
    phase_avg = jnp.mean(phase, axis=1, keepdims=True)
    offs = lax.broadcasted_iota(jnp.int32, (LBLK, SP), 1).astype(
        jnp.float32) - 16.0
    rel = offs * freq_avg + phase_avg
    rel_ref[...] = rel
    center = (i * LBLK).astype(jnp.float32) + lax.broadcasted_iota(
        jnp.int32, (LBLK, SP), 0).astype(jnp.float32)
    spos = center + rel
    sidx = jnp.clip(spos.astype(jnp.int32), 0, L - 1)
    i4 = sidx * NCH
    for c in range(NCH):
        idx_ref[:, c, :] = i4 + c


def _stage_a(x2, w_wave, b_wave2):
    return pl.pallas_call(
        _wave_body,
        grid=(L // LBLK,),
        in_specs=[
            pl.BlockSpec((LBLK, C), lambda i: (i, 0)),
            pl.BlockSpec((C, 2 * H), lambda i: (0, 0)),
            pl.BlockSpec((1, 2 * H), lambda i: (0, 0)),
        ],
        out_specs=[
            pl.BlockSpec((LBLK, SP), lambda i: (i, 0)),
            pl.BlockSpec((LBLK, NCH, SP), lambda i: (i, 0, 0)),
        ],
        out_shape=[
            jax.ShapeDtypeStruct((L, SP), jnp.float32),
            jax.ShapeDtypeStruct((L, NCH, SP), jnp.int32),
        ],
    )(x2, w_wave, b_wave2)


LBLKB = 128


def _weights_body(x_ref, wk_ref, bk_ref, rel_ref, w_ref):
    i = pl.program_id(1)
    km = _silu(jnp.dot(x_ref[...], wk_ref[...],
                       preferred_element_type=jnp.float32) + bk_ref[...])
    km3 = km.reshape(LBLKB, HPC, K)
    rel = rel_ref[...]
    center = (i * LBLKB).astype(jnp.float32) + lax.broadcasted_iota(
        jnp.int32, (LBLKB, SP), 0).astype(jnp.float32)
    spos = center + rel
    validf = ((spos >= 0.0) & (spos < float(L))).astype(jnp.float32)
    npos = jnp.clip((rel + MAX_RECEPTIVE) / (2.0 * MAX_RECEPTIVE), 0.0, 1.0)
    idxf = npos * (K - 1)
    ifl = jnp.clip(idxf.astype(jnp.int32), 0, K - 2)
    wce = idxf - ifl.astype(jnp.float32)
    wfl = 1.0 - wce
    pad_mask = (lax.broadcasted_iota(jnp.int32, (LBLKB, SP), 1)
                < S).astype(jnp.float32)
    validf = validf * pad_mask
    ifl3 = jnp.broadcast_to(ifl[:, None, :], (LBLKB, HPC, SP))
    kf = jnp.take_along_axis(km3, ifl3, axis=2)
    kc = jnp.take_along_axis(km3, ifl3 + 1, axis=2)
    kern = (wfl[:, None, :] * kf + wce[:, None, :] * kc) * validf[:, None, :]
    den = jnp.sum(kern, axis=2, keepdims=True) + 1e-8
    w_ref[:, 0, 0, :] = (kern / den).reshape(LBLKB, HPC * SP)


def _stage_b(x2, w_kernel, b_kernel2, rel):
    return pl.pallas_call(
        _weights_body,
        grid=(NCH, L // LBLKB),
        in_specs=[
            pl.BlockSpec((LBLKB, C), lambda j, i: (i, 0)),
            pl.BlockSpec((C, H * K // NCH), lambda j, i: (0, j)),
            pl.BlockSpec((1, H * K // NCH), lambda j, i: (0, j)),
            pl.BlockSpec((LBLKB, SP), lambda j, i: (i, 0)),
        ],
        out_specs=pl.BlockSpec((LBLKB, 1, 1, SP * HPC),
                               lambda j, i: (i, j, 0, 0)),
        out_shape=jax.ShapeDtypeStruct((L, NCH, 1, SP * HPC), jnp.float32),
    )(x2, w_kernel, b_kernel2, rel)


NTASK = TPW * NCH
SG = S


def _sc_compute(rows_v, w_v, acc_v):
    for h in range(HPC):
        def s_body(s, accs):
            wspl = plsc.load_gather(
                w_v, [jnp.full((16,), h * SP, jnp.int32) + s])
            return tuple(
                accs[j] + wspl * rows_v[s, pl.ds(h * D + j * 16, 16)]
                for j in range(8))
        accs = lax.fori_loop(
            0, SG, s_body,
            tuple(jnp.zeros((16,), jnp.float32) for _ in range(8)))
        for j in range(8):
            acc_v[pl.ds(h * D + j * 16, 16)] = accs[j]


def _sc_gather(xr, idx4, warr):
    mesh = plsc.VectorSubcoreMesh(core_axis_name="c", subcore_axis_name="s")

    @functools.partial(
        pl.kernel,
        mesh=mesh,
        out_type=jax.ShapeDtypeStruct((L, NCH, CHC), jnp.float32),
        scratch_types=[
            pltpu.VMEM((SG,), jnp.int32),
            pltpu.VMEM((SG,), jnp.int32),
            pltpu.VMEM((SG, CHC), jnp.float32),
            pltpu.VMEM((SG, CHC), jnp.float32),
            pltpu.VMEM((SP * HPC,), jnp.float32),
            pltpu.VMEM((SP * HPC,), jnp.float32),
            pltpu.VMEM((CHC,), jnp.float32),
            pltpu.VMEM((CHC,), jnp.float32),
            pltpu.SemaphoreType.DMA,
            pltpu.SemaphoreType.DMA,
            pltpu.SemaphoreType.DMA,
            pltpu.SemaphoreType.DMA,
            pltpu.SemaphoreType.DMA,
            pltpu.SemaphoreType.DMA,
            pltpu.SemaphoreType.DMA,
            pltpu.SemaphoreType.DMA,
        ],
        compiler_params=pltpu.CompilerParams(needs_layout_passes=False),
    )
    def body(xr_hbm, idx_hbm, w_hbm, out_hbm,
             idx_a, idx_b, rows0, rows1, w0, w1, acc0, acc1,
             s_ia, s_ib, s_r0, s_r1, s_w0, s_w1, s_s0, s_s1):
        wid = lax.axis_index("s") * 2 + lax.axis_index("c")
        base = wid * TPW

        def lc(task):
            return base + (task >> 2), task & 3

        l0, c0 = lc(0)
        pltpu.sync_copy(idx_hbm.at[l0, c0, pl.ds(0, SG)], idx_a)
        pltpu.async_copy(xr_hbm.at[idx_a], rows0, s_r0)
        l1, c1 = lc(1)
        pltpu.async_copy(idx_hbm.at[l1, c1, pl.ds(0, SG)], idx_b, s_ib)
        pltpu.async_copy(w_hbm.at[l0, c0, 0], w0, s_w0)

        def pair_body(p, carry):
            t_a = 2 * p
            t_b = t_a + 1
            t_c = t_a + 2
            la, ca = lc(t_a)
            lb, cb = lc(t_b)
            lcl, ccl = lc(jnp.minimum(t_c, NTASK - 1))
            pltpu.make_async_copy(idx_hbm.at[lb, cb, pl.ds(0, SG)], idx_b, s_ib).wait()
            pltpu.async_copy(xr_hbm.at[idx_b], rows1, s_r1)
            pltpu.async_copy(w_hbm.at[lb, cb, 0], w1, s_w1)
            pltpu.make_async_copy(w_hbm.at[la, ca, 0], w0, s_w0).wait()
            pltpu.make_async_copy(xr_hbm.at[idx_b], rows0, s_r0).wait()
            pltpu.async_copy(idx_hbm.at[lcl, ccl, pl.ds(0, SG)], idx_a, s_ia)

            @pl.when(p > 0)
            def _():
                pltpu.make_async_copy(acc0, out_hbm.at[la, ca], s_s0).wait()
            _sc_compute(rows0, w0, acc0)
            pltpu.async_copy(acc0, out_hbm.at[la, ca], s_s0)
            pltpu.make_async_copy(idx_hbm.at[lcl, ccl, pl.ds(0, SG)], idx_a, s_ia).wait()

            @pl.when(t_c < NTASK)
            def _():
                pltpu.async_copy(xr_hbm.at[idx_a], rows0, s_r0)
                pltpu.async_copy(w_hbm.at[lcl, ccl, 0], w0, s_w0)
            pltpu.make_async_copy(w_hbm.at[lb, cb, 0], w1, s_w1).wait()
            pltpu.make_async_copy(xr_hbm.at[idx_b], rows1, s_r1).wait()

            @pl.when(t_c < NTASK)
            def _():
                ld, cd = lc(jnp.minimum(t_c + 1, NTASK - 1))
                pltpu.async_copy(idx_hbm.at[ld, cd, pl.ds(0, SG)], idx_b, s_ib)

            @pl.when(p > 0)
            def _():
                pltpu.make_async_copy(acc1, out_hbm.at[lb, cb], s_s1).wait()
            _sc_compute(rows1, w1, acc1)
            pltpu.async_copy(acc1, out_hbm.at[lb, cb], s_s1)
            return carry

        lax.fori_loop(0, NTASK // 2, pair_body, 0)
        lf, cf = lc(NTASK - 2)
        pltpu.make_async_copy(acc0, out_hbm.at[lf, cf], s_s0).wait()
        lg, cg = lc(NTASK - 1)
        pltpu.make_async_copy(acc1, out_hbm.at[lg, cg], s_s1).wait()

    return body(xr, idx4, warr)


def _out_body(g_ref, wo_ref, o_ref):
    g = g_ref[...].astype(jnp.bfloat16)
    acc = jnp.dot(g, wo_ref[...], preferred_element_type=jnp.float32)
    o_ref[...] = _silu(acc)


def _stage_d(og, w_out):
    nj = 8
    return pl.pallas_call(
        _out_body,
        grid=(nj, L // LBLK),
        in_specs=[
            pl.BlockSpec((LBLK, C), lambda j, i: (i, 0)),
            pl.BlockSpec((C, C // nj), lambda j, i: (0, j)),
        ],
        out_specs=pl.BlockSpec((LBLK, C // nj), lambda j, i: (i, j)),
        out_shape=jax.ShapeDtypeStruct((L, C), jnp.float32),
    )(og, w_out)


def kernel(x, W_wave, b_wave, W_kernel, b_kernel, W_out):
    x2 = x[0]
    xr = x2.reshape(L * NCH, CHC)
    rel, idx4 = _stage_a(x2, W_wave, b_wave.reshape(1, -1))
    warr = _stage_b(x2, W_kernel, b_kernel.reshape(1, -1), rel)
    og = _sc_gather(xr, idx4, warr).reshape(L, C)
    y = _stage_d(og, W_out.astype(jnp.bfloat16))
    return y[None]
